# Initial kernel scaffold; baseline (speedup 1.0000x reference)
import functools

import jax
import jax.numpy as jnp
from jax import lax
from jax.experimental import pallas as pl
from jax.experimental.pallas import tpu as pltpu
from jax.experimental.pallas import tpu_sc as plsc

N_NODES = 10000
NC = 2
NS = 16
NW = NC * NS
CHUNK = 128
ACC_N = 10240
ROWS_PER_SUB = ACC_N // NS
OUT_PER_SUB = N_NODES // NS


def _zero_block(zbuf, d):
    zv = jnp.zeros((16,), jnp.float32)

    def body(r, _):
        for k in range(d // 16):
            zbuf[r, pl.ds(k * 16, 16)] = zv
        return 0

    lax.fori_loop(0, CHUNK, body, 0)


def _make_deg_kernel(cpw):
    mesh = plsc.VectorSubcoreMesh(core_axis_name="c", subcore_axis_name="s")

    @functools.partial(
        pl.kernel,
        out_type=jax.ShapeDtypeStruct((NC, N_NODES, 16), jnp.float32),
        mesh=mesh,
        scratch_types=[
            pltpu.VMEM((cpw, CHUNK), jnp.int32),
            pltpu.VMEM((CHUNK, 16), jnp.float32),
            pltpu.VMEM((CHUNK, 16), jnp.float32),
            pltpu.VMEM_SHARED((ACC_N, 16), jnp.float32),
        ],
    )
    def deg_kernel(dst_hbm, out_hbm, idx_v, ones_v, zbuf, acc_sh):
        c = lax.axis_index("c")
        s = lax.axis_index("s")
        wid = c * NS + s

        _zero_block(zbuf, 16)
        ov = jnp.ones((16,), jnp.float32)

        def fill_ones(r, _):
            ones_v[r, pl.ds(0, 16)] = ov
            return 0

        lax.fori_loop(0, CHUNK, fill_ones, 0)

        for j in range(ROWS_PER_SUB // CHUNK):
            pltpu.sync_copy(
                zbuf, acc_sh.at[pl.ds(s * ROWS_PER_SUB + j * CHUNK, CHUNK)])
        plsc.subcore_barrier()

        pltpu.sync_copy(dst_hbm.at[wid], idx_v)

        def chunk_body(i, _):
            pltpu.sync_copy(ones_v, acc_sh.at[idx_v.at[i]], add=True)
            return 0

        lax.fori_loop(0, cpw, chunk_body, 0)
        plsc.subcore_barrier()

        pltpu.sync_copy(acc_sh.at[pl.ds(s * OUT_PER_SUB, OUT_PER_SUB)],
                        out_hbm.at[c, pl.ds(s * OUT_PER_SUB, OUT_PER_SUB)])

    return deg_kernel


def _make_agg_kernel(cpw, d):
    mesh = plsc.VectorSubcoreMesh(core_axis_name="c", subcore_axis_name="s")

    @functools.partial(
        pl.kernel,
        out_type=jax.ShapeDtypeStruct((NC, N_NODES, d), jnp.float32),
        mesh=mesh,
        scratch_types=[
            pltpu.VMEM((cpw, CHUNK), jnp.int32),
            pltpu.VMEM((cpw, CHUNK), jnp.int32),
            pltpu.VMEM((CHUNK, d), jnp.float32),
            pltpu.VMEM((CHUNK, d), jnp.float32),
            pltpu.VMEM_SHARED((ACC_N, d), jnp.float32),
            pltpu.SemaphoreType.DMA,
        ],
    )
    def agg_kernel(table_hbm, src_hbm, dst_hbm, out_hbm,
                   src_v, dst_v, rows_v, zbuf, acc_sh, sem):
        c = lax.axis_index("c")
        s = lax.axis_index("s")
        wid = c * NS + s

        _zero_block(zbuf, d)
        for j in range(ROWS_PER_SUB // CHUNK):
            pltpu.sync_copy(
                zbuf, acc_sh.at[pl.ds(s * ROWS_PER_SUB + j * CHUNK, CHUNK)])
        plsc.subcore_barrier()

        pltpu.sync_copy(src_hbm.at[wid], src_v)
        pltpu.sync_copy(dst_hbm.at[wid], dst_v)

        def chunk_body(i, _):
            pltpu.async_copy(table_hbm.at[src_v.at[i]], rows_v, sem).wait()
            pltpu.sync_copy(rows_v, acc_sh.at[dst_v.at[i]], add=True)
            return 0

        lax.fori_loop(0, cpw, chunk_body, 0)
        plsc.subcore_barrier()

        pltpu.sync_copy(acc_sh.at[pl.ds(s * OUT_PER_SUB, OUT_PER_SUB)],
                        out_hbm.at[c, pl.ds(s * OUT_PER_SUB, OUT_PER_SUB)])

    return agg_kernel


def _tc_scale_matmul(x_ref, w1_ref, degp_ref, u_ref, dis_ref):
    deg = degp_ref[0] + degp_ref[1] + 1.0
    dis = lax.rsqrt(deg)
    h = jnp.dot(x_ref[...], w1_ref[...], preferred_element_type=jnp.float32)
    u_ref[...] = dis[:, 0:1] * h
    dis_ref[...] = dis


def _tc_mid(aggp_ref, u_ref, dis_ref, b1_ref, w2_ref, v_ref):
    dis = dis_ref[:, 0:1]
    y1 = jax.nn.relu(dis * (aggp_ref[0] + aggp_ref[1] + u_ref[...])
                     + b1_ref[...])
    h2 = jnp.dot(y1, w2_ref[...], preferred_element_type=jnp.float32)
    v_ref[...] = dis * h2


def _tc_out(aggp_ref, v_ref, dis_ref, b2_ref, o_ref):
    dis = dis_ref[:, 0:1]
    o_ref[...] = dis * (aggp_ref[0] + aggp_ref[1] + v_ref[...]) + b2_ref[...]


def kernel(x, edge_index, W1, b1, W2, b2):
    n = x.shape[0]
    e = edge_index.shape[1]
    h_dim = W1.shape[1]
    c_dim = W2.shape[1]
    d2 = 16

    cpw = -(-e // (NW * CHUNK))
    e_pad = NW * cpw * CHUNK
    pad = e_pad - e
    src = edge_index[0].astype(jnp.int32)
    dst = edge_index[1].astype(jnp.int32)
    if pad:
        fill = jnp.arange(pad, dtype=jnp.int32)
        src = jnp.concatenate([src, fill % n])
        dst = jnp.concatenate([dst, n + fill % (ACC_N - n)])
    src3 = src.reshape(NW, cpw, CHUNK)
    dst3 = dst.reshape(NW, cpw, CHUNK)

    W2p = jnp.zeros((h_dim, d2), jnp.float32).at[:, :c_dim].set(
        W2.astype(jnp.float32))
    b1r = b1.astype(jnp.float32).reshape(1, h_dim)
    b2r = jnp.zeros((1, d2), jnp.float32).at[0, :c_dim].set(
        b2.astype(jnp.float32))

    deg_parts = _make_deg_kernel(cpw)(dst3)

    u, dis = pl.pallas_call(
        _tc_scale_matmul,
        out_shape=(jax.ShapeDtypeStruct((n, h_dim), jnp.float32),
                   jax.ShapeDtypeStruct((n, 16), jnp.float32)),
    )(x.astype(jnp.float32), W1.astype(jnp.float32), deg_parts)

    agg1 = _make_agg_kernel(cpw, h_dim)(u, src3, dst3)

    v = pl.pallas_call(
        _tc_mid,
        out_shape=jax.ShapeDtypeStruct((n, d2), jnp.float32),
    )(agg1, u, dis, b1r, W2p)

    agg2 = _make_agg_kernel(cpw, d2)(v, src3, dst3)

    out = pl.pallas_call(
        _tc_out,
        out_shape=jax.ShapeDtypeStruct((n, d2), jnp.float32),
    )(agg2, v, dis, b2r)

    return out[:, :c_dim]


# speedup vs baseline: 43.2804x; 43.2804x over previous
import functools

import jax
import jax.numpy as jnp
from jax import lax
from jax.experimental import pallas as pl
from jax.experimental.pallas import tpu as pltpu
from jax.experimental.pallas import tpu_sc as plsc
from jax._src.config import enable_x64 as _enable_x64

NC = 2
NS = 16
NW = NC * NS
CHUNK = 128
ACC_N = 10240
RPS = ACC_N // NS

_SC_PARAMS = pltpu.CompilerParams(use_tc_tiling_on_sc=False)
_MESH = plsc.VectorSubcoreMesh(core_axis_name="c", subcore_axis_name="s",
                               num_cores=NC, num_subcores=NS)


def _zero_block(zbuf, d):
    zv = jnp.zeros((16,), jnp.float32)

    def body(r, _):
        for k in range(d // 16):
            zbuf[r, pl.ds(k * 16, 16)] = zv
        return 0

    lax.fori_loop(0, CHUNK, body, 0)


def _make_deg_kernel(cpw):
    @functools.partial(
        pl.kernel,
        out_type=jax.ShapeDtypeStruct((NC, ACC_N, 16), jnp.float32),
        mesh=_MESH,
        compiler_params=_SC_PARAMS,
        scratch_types=[
            pltpu.VMEM((cpw, CHUNK), jnp.int32),
            pltpu.VMEM((CHUNK, 16), jnp.float32),
            pltpu.VMEM((CHUNK, 16), jnp.float32),
            pltpu.VMEM_SHARED((ACC_N, 16), jnp.float32),
        ],
    )
    def deg_kernel(dst_hbm, out_hbm, idx_v, ones_v, zbuf, acc_sh):
        c = lax.axis_index("c")
        s = lax.axis_index("s")
        wid = c * NS + s

        _zero_block(zbuf, 16)
        ov = jnp.ones((16,), jnp.float32)

        def fill_ones(r, _):
            ones_v[r, pl.ds(0, 16)] = ov
            return 0

        lax.fori_loop(0, CHUNK, fill_ones, 0)

        for j in range(RPS // CHUNK):
            pltpu.sync_copy(zbuf,
                            acc_sh.at[pl.ds(s * RPS + j * CHUNK, CHUNK)])
        plsc.subcore_barrier()

        pltpu.sync_copy(dst_hbm.at[wid], idx_v)

        def chunk_body(i, _):
            pltpu.sync_copy(ones_v, acc_sh.at[idx_v.at[i]], add=True)
            return 0

        lax.fori_loop(0, cpw, chunk_body, 0)
        plsc.subcore_barrier()

        pltpu.sync_copy(acc_sh.at[pl.ds(s * RPS, RPS)],
                        out_hbm.at[c, pl.ds(s * RPS, RPS)])

    return deg_kernel


def _make_agg_kernel(cpw, d):
    @functools.partial(
        pl.kernel,
        out_type=jax.ShapeDtypeStruct((NC, ACC_N, d), jnp.float32),
        mesh=_MESH,
        compiler_params=_SC_PARAMS,
        scratch_types=[
            pltpu.VMEM((cpw, CHUNK), jnp.int32),
            pltpu.VMEM((cpw, CHUNK), jnp.int32),
            pltpu.VMEM((2, CHUNK, d), jnp.float32),
            pltpu.VMEM((CHUNK, d), jnp.float32),
            pltpu.VMEM_SHARED((ACC_N, d), jnp.float32),
            pltpu.SemaphoreType.DMA,
            pltpu.SemaphoreType.DMA,
        ],
    )
    def agg_kernel(table_hbm, src_hbm, dst_hbm, out_hbm,
                   src_v, dst_v, rows_v, zbuf, acc_sh, sem0, sem1):
        c = lax.axis_index("c")
        s = lax.axis_index("s")
        wid = c * NS + s

        _zero_block(zbuf, d)
        for j in range(RPS // CHUNK):
            pltpu.sync_copy(zbuf,
                            acc_sh.at[pl.ds(s * RPS + j * CHUNK, CHUNK)])
        plsc.subcore_barrier()

        pltpu.sync_copy(src_hbm.at[wid], src_v)
        pltpu.sync_copy(dst_hbm.at[wid], dst_v)

        sems = (sem0, sem1)
        bi = (0, 1)
        pltpu.async_copy(table_hbm.at[src_v.at[0]], rows_v.at[0], sem0)

        def outer(g, _):
            i0 = g * 2
            for b in range(2):
                i = i0 + b
                nxt = i + 1

                @pl.when(nxt < cpw)
                def _():
                    pltpu.async_copy(table_hbm.at[src_v.at[nxt]],
                                     rows_v.at[bi[1 - b]], sems[1 - b])

                pltpu.make_async_copy(table_hbm.at[src_v.at[i]],
                                      rows_v.at[bi[b]], sems[b]).wait()
                pltpu.sync_copy(rows_v.at[bi[b]], acc_sh.at[dst_v.at[i]],
                                add=True)
            return 0

        lax.fori_loop(0, cpw // 2, outer, 0)
        plsc.subcore_barrier()

        pltpu.sync_copy(acc_sh.at[pl.ds(s * RPS, RPS)],
                        out_hbm.at[c, pl.ds(s * RPS, RPS)])

    return agg_kernel


def _tc_scale_matmul(x_ref, w1_ref, degp_ref, u_ref, dis_ref):
    n = x_ref.shape[0]
    deg = degp_ref[0, :n] + degp_ref[1, :n] + 1.0
    dis = lax.rsqrt(deg)
    h = jnp.dot(x_ref[...], w1_ref[...], preferred_element_type=jnp.float32)
    u_ref[...] = dis[:, 0:1] * h
    dis_ref[...] = dis


def _tc_mid(aggp_ref, u_ref, dis_ref, b1_ref, w2_ref, v_ref):
    n = u_ref.shape[0]
    dis = dis_ref[:, 0:1]
    y1 = jax.nn.relu(dis * (aggp_ref[0, :n] + aggp_ref[1, :n] + u_ref[...])
                     + b1_ref[...])
    h2 = jnp.dot(y1, w2_ref[...], preferred_element_type=jnp.float32)
    v_ref[...] = dis * h2


def _tc_out(aggp_ref, v_ref, dis_ref, b2_ref, o_ref):
    n = v_ref.shape[0]
    dis = dis_ref[:, 0:1]
    o_ref[...] = (dis * (aggp_ref[0, :n] + aggp_ref[1, :n] + v_ref[...])
                  + b2_ref[...])


def kernel(x, edge_index, W1, b1, W2, b2):
    with _enable_x64(False):
        return _kernel_impl(x, edge_index, W1, b1, W2, b2)


def _kernel_impl(x, edge_index, W1, b1, W2, b2):
    n = x.shape[0]
    e = edge_index.shape[1]
    h_dim = W1.shape[1]
    c_dim = W2.shape[1]
    d2 = 16

    cpw = -(-e // (NW * CHUNK))
    if cpw % 2:
        cpw += 1
    e_pad = NW * cpw * CHUNK
    pad = e_pad - e
    src = edge_index[0].astype(jnp.int32)
    dst = edge_index[1].astype(jnp.int32)
    if pad:
        fill = jnp.arange(pad, dtype=jnp.int32)
        src = jnp.concatenate([src, fill % n])
        dst = jnp.concatenate([dst, n + fill % (ACC_N - n)])
    src3 = src.reshape(NW, cpw, CHUNK)
    dst3 = dst.reshape(NW, cpw, CHUNK)

    W2p = jnp.zeros((h_dim, d2), jnp.float32).at[:, :c_dim].set(
        W2.astype(jnp.float32))
    b1r = b1.astype(jnp.float32).reshape(1, h_dim)
    b2r = jnp.zeros((1, d2), jnp.float32).at[0, :c_dim].set(
        b2.astype(jnp.float32))

    deg_parts = _make_deg_kernel(cpw)(dst3)

    u, dis = pl.pallas_call(
        _tc_scale_matmul,
        out_shape=(jax.ShapeDtypeStruct((n, h_dim), jnp.float32),
                   jax.ShapeDtypeStruct((n, 16), jnp.float32)),
    )(x.astype(jnp.float32), W1.astype(jnp.float32), deg_parts)

    agg1 = _make_agg_kernel(cpw, h_dim)(u, src3, dst3)

    v = pl.pallas_call(
        _tc_mid,
        out_shape=jax.ShapeDtypeStruct((n, d2), jnp.float32),
    )(agg1, u, dis, b1r, W2p)

    agg2 = _make_agg_kernel(cpw, d2)(v, src3, dst3)

    out = pl.pallas_call(
        _tc_out,
        out_shape=jax.ShapeDtypeStruct((n, d2), jnp.float32),
    )(agg2, v, dis, b2r)

    return out[:, :c_dim]


# speedup vs baseline: 48.7345x; 1.1260x over previous
import functools

import jax
import jax.numpy as jnp
from jax import lax
from jax.experimental import pallas as pl
from jax.experimental.pallas import tpu as pltpu
from jax.experimental.pallas import tpu_sc as plsc
from jax._src.config import enable_x64 as _enable_x64

NC = 2
NS = 16
NW = NC * NS
CHUNK = 256
ZB = 128
ACC_N = 10240
RPS = ACC_N // NS

_SC_PARAMS = pltpu.CompilerParams(use_tc_tiling_on_sc=False)
_MESH = plsc.VectorSubcoreMesh(core_axis_name="c", subcore_axis_name="s",
                               num_cores=NC, num_subcores=NS)


def _zero_block(zbuf, d):
    zv = jnp.zeros((16,), jnp.float32)

    def body(r, _):
        for k in range(d // 16):
            zbuf[r, pl.ds(k * 16, 16)] = zv
        return 0

    lax.fori_loop(0, ZB, body, 0)


def _make_deg_kernel(cpw):
    @functools.partial(
        pl.kernel,
        out_type=jax.ShapeDtypeStruct((NC, ACC_N, 16), jnp.float32),
        mesh=_MESH,
        compiler_params=_SC_PARAMS,
        scratch_types=[
            pltpu.VMEM((cpw, CHUNK), jnp.int32),
            pltpu.VMEM((CHUNK, 16), jnp.float32),
            pltpu.VMEM((ZB, 16), jnp.float32),
            pltpu.VMEM_SHARED((ACC_N, 16), jnp.float32),
        ],
    )
    def deg_kernel(dst_hbm, out_hbm, idx_v, ones_v, zbuf, acc_sh):
        c = lax.axis_index("c")
        s = lax.axis_index("s")
        wid = c * NS + s

        _zero_block(zbuf, 16)
        ov = jnp.ones((16,), jnp.float32)

        def fill_ones(r, _):
            ones_v[r, pl.ds(0, 16)] = ov
            return 0

        lax.fori_loop(0, CHUNK, fill_ones, 0)

        for j in range(RPS // ZB):
            pltpu.sync_copy(zbuf,
                            acc_sh.at[pl.ds(s * RPS + j * ZB, ZB)])
        plsc.subcore_barrier()

        pltpu.sync_copy(dst_hbm.at[wid], idx_v)

        def chunk_body(i, _):
            pltpu.sync_copy(ones_v, acc_sh.at[idx_v.at[i]], add=True)
            return 0

        lax.fori_loop(0, cpw, chunk_body, 0)
        plsc.subcore_barrier()

        pltpu.sync_copy(acc_sh.at[pl.ds(s * RPS, RPS)],
                        out_hbm.at[c, pl.ds(s * RPS, RPS)])

    return deg_kernel


def _make_agg_kernel(cpw, d):
    @functools.partial(
        pl.kernel,
        out_type=jax.ShapeDtypeStruct((NC, ACC_N, d), jnp.float32),
        mesh=_MESH,
        compiler_params=_SC_PARAMS,
        scratch_types=[
            pltpu.VMEM((cpw, CHUNK), jnp.int32),
            pltpu.VMEM((cpw, CHUNK), jnp.int32),
            pltpu.VMEM((2, CHUNK, d), jnp.float32),
            pltpu.VMEM((ZB, d), jnp.float32),
            pltpu.VMEM_SHARED((ACC_N, d), jnp.float32),
            pltpu.SemaphoreType.DMA,
            pltpu.SemaphoreType.DMA,
        ],
    )
    def agg_kernel(table_hbm, src_hbm, dst_hbm, out_hbm,
                   src_v, dst_v, rows_v, zbuf, acc_sh, sem0, sem1):
        c = lax.axis_index("c")
        s = lax.axis_index("s")
        wid = c * NS + s

        _zero_block(zbuf, d)
        for j in range(RPS // ZB):
            pltpu.sync_copy(zbuf,
                            acc_sh.at[pl.ds(s * RPS + j * ZB, ZB)])
        plsc.subcore_barrier()

        pltpu.sync_copy(src_hbm.at[wid], src_v)
        pltpu.sync_copy(dst_hbm.at[wid], dst_v)

        sems = (sem0, sem1)
        bi = (0, 1)
        pltpu.async_copy(table_hbm.at[src_v.at[0]], rows_v.at[0], sem0)

        def outer(g, _):
            i0 = g * 2
            for b in range(2):
                i = i0 + b
                nxt = i + 1

                @pl.when(nxt < cpw)
                def _():
                    pltpu.async_copy(table_hbm.at[src_v.at[nxt]],
                                     rows_v.at[bi[1 - b]], sems[1 - b])

                pltpu.make_async_copy(table_hbm.at[src_v.at[i]],
                                      rows_v.at[bi[b]], sems[b]).wait()
                pltpu.sync_copy(rows_v.at[bi[b]], acc_sh.at[dst_v.at[i]],
                                add=True)
            return 0

        lax.fori_loop(0, cpw // 2, outer, 0)
        plsc.subcore_barrier()

        pltpu.sync_copy(acc_sh.at[pl.ds(s * RPS, RPS)],
                        out_hbm.at[c, pl.ds(s * RPS, RPS)])

    return agg_kernel


def _tc_scale_matmul(x_ref, w1_ref, degp_ref, u_ref, dis_ref):
    n = x_ref.shape[0]
    deg = degp_ref[0, :n] + degp_ref[1, :n] + 1.0
    dis = lax.rsqrt(deg)
    h = jnp.dot(x_ref[...], w1_ref[...], preferred_element_type=jnp.float32)
    u_ref[...] = dis[:, 0:1] * h
    dis_ref[...] = dis


def _tc_mid(aggp_ref, u_ref, dis_ref, b1_ref, w2_ref, v_ref):
    n = u_ref.shape[0]
    dis = dis_ref[:, 0:1]
    y1 = jax.nn.relu(dis * (aggp_ref[0, :n] + aggp_ref[1, :n] + u_ref[...])
                     + b1_ref[...])
    h2 = jnp.dot(y1, w2_ref[...], preferred_element_type=jnp.float32)
    v_ref[...] = dis * h2


def _tc_out(aggp_ref, v_ref, dis_ref, b2_ref, o_ref):
    n = v_ref.shape[0]
    dis = dis_ref[:, 0:1]
    o_ref[...] = (dis * (aggp_ref[0, :n] + aggp_ref[1, :n] + v_ref[...])
                  + b2_ref[...])


def kernel(x, edge_index, W1, b1, W2, b2):
    with _enable_x64(False):
        return _kernel_impl(x, edge_index, W1, b1, W2, b2)


def _kernel_impl(x, edge_index, W1, b1, W2, b2):
    n = x.shape[0]
    e = edge_index.shape[1]
    h_dim = W1.shape[1]
    c_dim = W2.shape[1]
    d2 = 16

    cpw = -(-e // (NW * CHUNK))
    if cpw % 2:
        cpw += 1
    e_pad = NW * cpw * CHUNK
    pad = e_pad - e
    src = edge_index[0].astype(jnp.int32)
    dst = edge_index[1].astype(jnp.int32)
    if pad:
        fill = jnp.arange(pad, dtype=jnp.int32)
        src = jnp.concatenate([src, fill % n])
        dst = jnp.concatenate([dst, n + fill % (ACC_N - n)])
    src3 = src.reshape(NW, cpw, CHUNK)
    dst3 = dst.reshape(NW, cpw, CHUNK)

    W2p = jnp.zeros((h_dim, d2), jnp.float32).at[:, :c_dim].set(
        W2.astype(jnp.float32))
    b1r = b1.astype(jnp.float32).reshape(1, h_dim)
    b2r = jnp.zeros((1, d2), jnp.float32).at[0, :c_dim].set(
        b2.astype(jnp.float32))

    deg_parts = _make_deg_kernel(cpw)(dst3)

    u, dis = pl.pallas_call(
        _tc_scale_matmul,
        out_shape=(jax.ShapeDtypeStruct((n, h_dim), jnp.float32),
                   jax.ShapeDtypeStruct((n, 16), jnp.float32)),
    )(x.astype(jnp.float32), W1.astype(jnp.float32), deg_parts)

    agg1 = _make_agg_kernel(cpw, h_dim)(u, src3, dst3)

    v = pl.pallas_call(
        _tc_mid,
        out_shape=jax.ShapeDtypeStruct((n, d2), jnp.float32),
    )(agg1, u, dis, b1r, W2p)

    agg2 = _make_agg_kernel(cpw, d2)(v, src3, dst3)

    out = pl.pallas_call(
        _tc_out,
        out_shape=jax.ShapeDtypeStruct((n, d2), jnp.float32),
    )(agg2, v, dis, b2r)

    return out[:, :c_dim]


# speedup vs baseline: 50.3678x; 1.0335x over previous
import functools

import jax
import jax.numpy as jnp
from jax import lax
from jax.experimental import pallas as pl
from jax.experimental.pallas import tpu as pltpu
from jax.experimental.pallas import tpu_sc as plsc
from jax._src.config import enable_x64 as _enable_x64

NC = 2
NS = 16
NW = NC * NS
CHUNK = 256
ZB = 128
ACC_N = 10240
RPS = ACC_N // NS

_SC_PARAMS = pltpu.CompilerParams(use_tc_tiling_on_sc=False)
_MESH = plsc.VectorSubcoreMesh(core_axis_name="c", subcore_axis_name="s",
                               num_cores=NC, num_subcores=NS)


def _zero_block(zbuf, d):
    zv = jnp.zeros((16,), jnp.float32)

    def body(r, _):
        for k in range(d // 16):
            zbuf[r, pl.ds(k * 16, 16)] = zv
        return 0

    lax.fori_loop(0, ZB, body, 0)


def _make_deg_kernel(cpw, chunk):
    @functools.partial(
        pl.kernel,
        out_type=jax.ShapeDtypeStruct((NC, ACC_N, 16), jnp.float32),
        mesh=_MESH,
        compiler_params=_SC_PARAMS,
        scratch_types=[
            pltpu.VMEM((cpw, chunk), jnp.int32),
            pltpu.VMEM((chunk, 16), jnp.float32),
            pltpu.VMEM((ZB, 16), jnp.float32),
            pltpu.VMEM_SHARED((ACC_N, 16), jnp.float32),
        ],
    )
    def deg_kernel(dst_hbm, out_hbm, idx_v, ones_v, zbuf, acc_sh):
        c = lax.axis_index("c")
        s = lax.axis_index("s")
        wid = c * NS + s

        _zero_block(zbuf, 16)
        ov = jnp.ones((16,), jnp.float32)

        def fill_ones(r, _):
            ones_v[r, pl.ds(0, 16)] = ov
            return 0

        lax.fori_loop(0, chunk, fill_ones, 0)

        for j in range(RPS // ZB):
            pltpu.sync_copy(zbuf,
                            acc_sh.at[pl.ds(s * RPS + j * ZB, ZB)])
        plsc.subcore_barrier()

        pltpu.sync_copy(dst_hbm.at[wid], idx_v)

        def chunk_body(i, _):
            pltpu.sync_copy(ones_v, acc_sh.at[idx_v.at[i]], add=True)
            return 0

        lax.fori_loop(0, cpw, chunk_body, 0)
        plsc.subcore_barrier()

        pltpu.sync_copy(acc_sh.at[pl.ds(s * RPS, RPS)],
                        out_hbm.at[c, pl.ds(s * RPS, RPS)])

    return deg_kernel


def _make_agg_kernel(cpw, d, chunk):
    @functools.partial(
        pl.kernel,
        out_type=jax.ShapeDtypeStruct((NC, ACC_N, d), jnp.float32),
        mesh=_MESH,
        compiler_params=_SC_PARAMS,
        scratch_types=[
            pltpu.VMEM((cpw, chunk), jnp.int32),
            pltpu.VMEM((cpw, chunk), jnp.int32),
            pltpu.VMEM((2, chunk, d), jnp.float32),
            pltpu.VMEM((ZB, d), jnp.float32),
            pltpu.VMEM_SHARED((ACC_N, d), jnp.float32),
            pltpu.SemaphoreType.DMA,
            pltpu.SemaphoreType.DMA,
        ],
    )
    def agg_kernel(table_hbm, src_hbm, dst_hbm, out_hbm,
                   src_v, dst_v, rows_v, zbuf, acc_sh, sem0, sem1):
        c = lax.axis_index("c")
        s = lax.axis_index("s")
        wid = c * NS + s

        _zero_block(zbuf, d)
        for j in range(RPS // ZB):
            pltpu.sync_copy(zbuf,
                            acc_sh.at[pl.ds(s * RPS + j * ZB, ZB)])
        plsc.subcore_barrier()

        pltpu.sync_copy(src_hbm.at[wid], src_v)
        pltpu.sync_copy(dst_hbm.at[wid], dst_v)

        sems = (sem0, sem1)
        bi = (0, 1)
        pltpu.async_copy(table_hbm.at[src_v.at[0]], rows_v.at[0], sem0)

        def outer(g, _):
            i0 = g * 2
            for b in range(2):
                i = i0 + b
                nxt = i + 1

                @pl.when(nxt < cpw)
                def _():
                    pltpu.async_copy(table_hbm.at[src_v.at[nxt]],
                                     rows_v.at[bi[1 - b]], sems[1 - b])

                pltpu.make_async_copy(table_hbm.at[src_v.at[i]],
                                      rows_v.at[bi[b]], sems[b]).wait()
                pltpu.sync_copy(rows_v.at[bi[b]], acc_sh.at[dst_v.at[i]],
                                add=True)
            return 0

        lax.fori_loop(0, cpw // 2, outer, 0)
        plsc.subcore_barrier()

        pltpu.sync_copy(acc_sh.at[pl.ds(s * RPS, RPS)],
                        out_hbm.at[c, pl.ds(s * RPS, RPS)])

    return agg_kernel


def _tc_scale_matmul(x_ref, w1_ref, degp_ref, u_ref, dis_ref):
    n = x_ref.shape[0]
    deg = degp_ref[0, :n] + degp_ref[1, :n] + 1.0
    dis = lax.rsqrt(deg)
    h = jnp.dot(x_ref[...], w1_ref[...], preferred_element_type=jnp.float32)
    u_ref[...] = dis[:, 0:1] * h
    dis_ref[...] = dis


def _tc_mid(aggp_ref, u_ref, dis_ref, b1_ref, w2_ref, v_ref):
    n = u_ref.shape[0]
    dis = dis_ref[:, 0:1]
    y1 = jax.nn.relu(dis * (aggp_ref[0, :n] + aggp_ref[1, :n] + u_ref[...])
                     + b1_ref[...])
    h2 = jnp.dot(y1, w2_ref[...], preferred_element_type=jnp.float32)
    v_ref[...] = dis * h2


def _tc_out(aggp_ref, v_ref, dis_ref, b2_ref, o_ref):
    n = v_ref.shape[0]
    dis = dis_ref[:, 0:1]
    o_ref[...] = (dis * (aggp_ref[0, :n] + aggp_ref[1, :n] + v_ref[...])
                  + b2_ref[...])


def kernel(x, edge_index, W1, b1, W2, b2):
    with _enable_x64(False):
        return _kernel_impl(x, edge_index, W1, b1, W2, b2)


def _kernel_impl(x, edge_index, W1, b1, W2, b2):
    n = x.shape[0]
    e = edge_index.shape[1]
    h_dim = W1.shape[1]
    c_dim = W2.shape[1]
    d2 = 16

    ch_w = CHUNK
    ch_n = 4 * CHUNK
    quant = NW * ch_n * 2
    e_pad = -(-e // quant) * quant
    cpw_w = e_pad // (NW * ch_w)
    cpw_n = e_pad // (NW * ch_n)
    pad = e_pad - e
    src = edge_index[0].astype(jnp.int32)
    dst = edge_index[1].astype(jnp.int32)
    if pad:
        fill = jnp.arange(pad, dtype=jnp.int32)
        src = jnp.concatenate([src, fill % n])
        dst = jnp.concatenate([dst, n + fill % (ACC_N - n)])
    src3w = src.reshape(NW, cpw_w, ch_w)
    dst3w = dst.reshape(NW, cpw_w, ch_w)
    src3n = src.reshape(NW, cpw_n, ch_n)
    dst3n = dst.reshape(NW, cpw_n, ch_n)

    W2p = jnp.zeros((h_dim, d2), jnp.float32).at[:, :c_dim].set(
        W2.astype(jnp.float32))
    b1r = b1.astype(jnp.float32).reshape(1, h_dim)
    b2r = jnp.zeros((1, d2), jnp.float32).at[0, :c_dim].set(
        b2.astype(jnp.float32))

    deg_parts = _make_deg_kernel(cpw_n, ch_n)(dst3n)

    u, dis = pl.pallas_call(
        _tc_scale_matmul,
        out_shape=(jax.ShapeDtypeStruct((n, h_dim), jnp.float32),
                   jax.ShapeDtypeStruct((n, 16), jnp.float32)),
    )(x.astype(jnp.float32), W1.astype(jnp.float32), deg_parts)

    agg1 = _make_agg_kernel(cpw_w, h_dim, ch_w)(u, src3w, dst3w)

    v = pl.pallas_call(
        _tc_mid,
        out_shape=jax.ShapeDtypeStruct((n, d2), jnp.float32),
    )(agg1, u, dis, b1r, W2p)

    agg2 = _make_agg_kernel(cpw_n, d2, ch_n)(v, src3n, dst3n)

    out = pl.pallas_call(
        _tc_out,
        out_shape=jax.ShapeDtypeStruct((n, d2), jnp.float32),
    )(agg2, v, dis, b2r)

    return out[:, :c_dim]
